# baseline (speedup 1.0000x reference)
import functools
import jax
import jax.numpy as jnp
from jax import lax
from jax.experimental import pallas as pl
from jax.experimental.pallas import tpu as pltpu
from jax.experimental.pallas import tpu_sc as plsc

F32 = jnp.float32
I32 = jnp.int32

NSEG = 196
NTOK = 392
NTOKP = 512
C1 = 64
C2 = 384
HID = 512
HF = 112
WF = 112
NPIX = HF * WF
PH = 114
PW = 128
PFLAT = PH * PW
CNT_BN = 2.0 * NPIX


def _k1_body(p_ref, w_ref, b_ref, x1_ref, s_ref):
    b = pl.program_id(0)
    x = jnp.dot(w_ref[...], p_ref[0], preferred_element_type=F32) + b_ref[...]
    x1_ref[0] = x
    ssum = jnp.sum(x, axis=1, keepdims=True)
    ssq = jnp.sum(x * x, axis=1, keepdims=True)
    blk = jnp.concatenate([ssum, ssq, jnp.zeros((C1, 126), F32)], axis=1)

    @pl.when(b == 0)
    def _():
        s_ref[...] = blk

    @pl.when(b != 0)
    def _():
        s_ref[...] = s_ref[...] + blk


def _run_k1(p, w1r, b1):
    B = p.shape[0]
    return pl.pallas_call(
        _k1_body,
        grid=(B,),
        in_specs=[
            pl.BlockSpec((1, 147, NPIX), lambda b: (b, 0, 0)),
            pl.BlockSpec((C1, 147), lambda b: (0, 0)),
            pl.BlockSpec((C1, 1), lambda b: (0, 0)),
        ],
        out_specs=[
            pl.BlockSpec((1, C1, NPIX), lambda b: (b, 0, 0)),
            pl.BlockSpec((C1, 128), lambda b: (0, 0)),
        ],
        out_shape=[
            jax.ShapeDtypeStruct((B, C1, NPIX), F32),
            jax.ShapeDtypeStruct((C1, 128), F32),
        ],
    )(p, w1r, b1)


def _k2a_body(x_ref, s_ref, g_ref, bb_ref, o_ref):
    ssum = s_ref[:, 0:1]
    ssq = s_ref[:, 1:2]
    mean = ssum / CNT_BN
    var = ssq / CNT_BN - mean * mean
    sc = g_ref[...] * jax.lax.rsqrt(var + 1e-5)
    sh = bb_ref[...] - mean * sc
    y = jnp.maximum(x_ref[0] * sc[:, :, None] + sh[:, :, None], 0.0)
    zc_l = jnp.zeros((C1, HF, 1), F32)
    zc_r = jnp.zeros((C1, HF, PW - WF - 1), F32)
    zrow = jnp.zeros((C1, 1, PW), F32)
    mid = jnp.concatenate([zc_l, y, zc_r], axis=2)
    o_ref[0] = jnp.concatenate([zrow, mid, zrow], axis=1)


def _run_k2a(x1, stats1, g1, b1g):
    B = x1.shape[0]
    x1v = x1.reshape(B, C1, HF, WF)
    return pl.pallas_call(
        _k2a_body,
        grid=(B,),
        in_specs=[
            pl.BlockSpec((1, C1, HF, WF), lambda b: (b, 0, 0, 0)),
            pl.BlockSpec((C1, 128), lambda b: (0, 0)),
            pl.BlockSpec((C1, 1), lambda b: (0, 0)),
            pl.BlockSpec((C1, 1), lambda b: (0, 0)),
        ],
        out_specs=pl.BlockSpec((1, C1, PH, PW), lambda b: (b, 0, 0, 0)),
        out_shape=jax.ShapeDtypeStruct((B, C1, PH, PW), F32),
    )(x1v, stats1, g1, b1g)


_TAPS = [(dh, dw) for dh in (-1, 0, 1) for dw in (-1, 0, 1)]


def _k2b_body(a_ref, w_ref, b2_ref, m_ref, o_ref, s_ref):
    b = pl.program_id(1)
    a = a_ref[0]
    o_ref[0] = jnp.zeros((128, PFLAT), F32) + b2_ref[...]
    for t, (dh, dw) in enumerate(_TAPS):
        d = dh * PW + dw
        lo_in = max(d, 0)
        hi_in = PFLAT + min(d, 0)
        lo_o = max(-d, 0)
        hi_o = lo_o + (hi_in - lo_in)
        o_ref[0, :, lo_o:hi_o] += jnp.dot(
            w_ref[t], a[:, lo_in:hi_in], preferred_element_type=F32)
    y = o_ref[0]
    m = m_ref[0:1, :]
    ssum = jnp.sum(y * m, axis=1, keepdims=True)
    ssq = jnp.sum(y * y * m, axis=1, keepdims=True)
    blk = jnp.concatenate([ssum, ssq, jnp.zeros((128, 126), F32)], axis=1)

    @pl.when(b == 0)
    def _():
        s_ref[...] = blk

    @pl.when(b != 0)
    def _():
        s_ref[...] = s_ref[...] + blk


def _run_k2b(a1f, w2r, b2, maskv):
    B = a1f.shape[0]
    return pl.pallas_call(
        _k2b_body,
        grid=(C2 // 128, B),
        in_specs=[
            pl.BlockSpec((1, C1, PFLAT), lambda c, b: (b, 0, 0)),
            pl.BlockSpec((9, 128, C1), lambda c, b: (0, c, 0)),
            pl.BlockSpec((128, 1), lambda c, b: (c, 0)),
            pl.BlockSpec((8, PFLAT), lambda c, b: (0, 0)),
        ],
        out_specs=[
            pl.BlockSpec((1, 128, PFLAT), lambda c, b: (b, c, 0)),
            pl.BlockSpec((128, 128), lambda c, b: (c, 0)),
        ],
        out_shape=[
            jax.ShapeDtypeStruct((B, C2, PFLAT), F32),
            jax.ShapeDtypeStruct((C2, 128), F32),
        ],
    )(a1f, w2r, b2, maskv)


def _k3a_body(seg_ref, segds_ref, cent_ref, win_ref,
              segv, accv, sumv, wsegv, wtv, shared, sem):
    cid = lax.axis_index("c")
    sid = lax.axis_index("s")
    wid = sid * 2 + cid
    lanes = lax.broadcasted_iota(I32, (16,), 0)
    zero16 = jnp.zeros((16,), F32)

    def _z(i, _):
        for l in range(16):
            accv[l, pl.ds(i * 16, 16)] = zero16
        return 0
    lax.fori_loop(0, 1536 // 16, _z, 0)

    base = wid * 3136
    pltpu.sync_copy(seg_ref.at[pl.ds(base, 3136)], segv)

    def _cent(i, _):
        sv = segv[pl.ds(i * 16, 16)]
        pg = base + i * 16 + lanes
        bb = pg // (224 * 224)
        p = pg - bb * (224 * 224)
        yy = p // 224
        xx = p - yy * 224
        gid = bb * NSEG + sv
        plsc.addupdate_scatter(accv, [lanes, gid], xx.astype(F32))
        plsc.addupdate_scatter(accv, [lanes, gid + NTOKP], yy.astype(F32))
        plsc.addupdate_scatter(accv, [lanes, gid + 2 * NTOKP],
                               jnp.ones((16,), F32))
        return 0
    lax.fori_loop(0, 3136 // 16, _cent, 0)

    def _fold(i, _):
        acc = zero16
        for l in range(16):
            acc = acc + accv[l, pl.ds(i * 16, 16)]
        sumv[pl.ds(i * 16, 16)] = acc
        return 0
    lax.fori_loop(0, 1536 // 16, _fold, 0)

    @pl.when(jnp.logical_and(cid == 1, sid == 15))
    def _():
        pltpu.sync_copy(segds_ref, wsegv)
        neg = jnp.zeros((16,), I32) - 1

        def _wz(i, _):
            for l in range(16):
                wtv[l, pl.ds(i * 16, 16)] = neg
            return 0
        lax.fori_loop(0, NTOKP // 16, _wz, 0)

        def _win(i, _):
            sv = wsegv[pl.ds(i * 16, 16)]
            iflat = i * 16 + lanes
            bb = iflat // NPIX
            gid = bb * NSEG + sv
            plsc.store_scatter(wtv, [lanes, gid], iflat)
            return 0
        lax.fori_loop(0, (2 * NPIX) // 16, _win, 0)

        def _wfold(i, _):
            acc = neg
            for l in range(16):
                acc = jnp.maximum(acc, wtv[l, pl.ds(i * 16, 16)])
            wsegv[pl.ds(i * 16, 16)] = acc
            return 0
        lax.fori_loop(0, NTOKP // 16, _wfold, 0)
        pltpu.sync_copy(wsegv.at[pl.ds(0, NTOKP)], win_ref)

    pltpu.sync_copy(sumv, shared.at[sid])
    plsc.subcore_barrier()

    @pl.when(sid == 0)
    def _():
        pltpu.sync_copy(shared, accv)

        def _red(i, _):
            acc = zero16
            for s in range(16):
                acc = acc + accv[s, pl.ds(i * 16, 16)]
            sumv[pl.ds(i * 16, 16)] = acc
            return 0
        lax.fori_loop(0, 1536 // 16, _red, 0)
        pltpu.sync_copy(sumv, cent_ref.at[cid])


def _run_k3a(seg_flat, segds_flat):
    mesh = plsc.VectorSubcoreMesh(core_axis_name="c", subcore_axis_name="s")
    return pl.kernel(
        _k3a_body,
        compiler_params=pltpu.CompilerParams(needs_layout_passes=False),
        out_type=[
            jax.ShapeDtypeStruct((2, 1536), F32),
            jax.ShapeDtypeStruct((NTOKP,), I32),
        ],
        mesh=mesh,
        scratch_types=[
            pltpu.VMEM((3136,), I32),
            pltpu.VMEM((16, 1536), F32),
            pltpu.VMEM((1536,), F32),
            pltpu.VMEM((2 * NPIX,), I32),
            pltpu.VMEM((16, NTOKP), I32),
            pltpu.VMEM_SHARED((16, 1536), F32),
            pltpu.SemaphoreType.DMA,
        ],
    )(seg_flat, segds_flat)


def _k3b_body(x2v_ref, win_ref, rows_ref, meta_ref,
              winv, staged, rowbuf, metabuf, sem):
    cid = lax.axis_index("c")
    sid = lax.axis_index("s")
    wid = sid * 2 + cid
    lanes = lax.broadcasted_iota(I32, (16,), 0)
    pltpu.sync_copy(win_ref, winv)

    def _row(k, _):
        r = (wid + k * 32) % NTOK
        i = plsc.load_gather(winv, [jnp.zeros((16,), I32) + r])
        bb = i // NPIX
        n = i - bb * NPIX
        q0 = n * C2
        c20 = q0 // NPIX
        p0 = q0 - c20 * NPIX
        w0 = p0 % WF
        qs = q0 + jnp.minimum(
            jnp.maximum((WF - w0) + (lanes - 1) * WF, 0), C2 - 1)
        c2s = qs // NPIX
        ps = qs - c2s * NPIX
        hs = ps // WF
        hrow = (bb * C2 + c2s) * PH + hs + 1
        pltpu.async_copy(x2v_ref.at[hrow], staged, sem).wait()

        def _pack(j24, _):
            jv = j24 * 16 + lanes
            slot = (w0 + jv) // WF
            wj = (w0 + jv) - slot * WF
            rowbuf[pl.ds(j24 * 16, 16)] = plsc.load_gather(
                staged, [slot, wj + 1])
            return 0
        lax.fori_loop(0, C2 // 16, _pack, 0)

        pltpu.sync_copy(rowbuf, rows_ref.at[r])
        tsplit = (c20 + 1) * NPIX - q0
        mv = (jnp.where(lanes == 0, c20, 0) +
              jnp.where(lanes == 1, tsplit, 0)).astype(F32)
        metabuf[...] = mv
        pltpu.sync_copy(metabuf, meta_ref.at[r])
        return 0
    lax.fori_loop(0, 13, _row, 0)


def _run_k3b(x2v, winners):
    mesh = plsc.VectorSubcoreMesh(core_axis_name="c", subcore_axis_name="s")
    return pl.kernel(
        _k3b_body,
        compiler_params=pltpu.CompilerParams(needs_layout_passes=False),
        out_type=[
            jax.ShapeDtypeStruct((NTOK, C2), F32),
            jax.ShapeDtypeStruct((NTOK, 16), F32),
        ],
        mesh=mesh,
        scratch_types=[
            pltpu.VMEM((NTOKP,), I32),
            pltpu.VMEM((16, PW), F32),
            pltpu.VMEM((C2,), F32),
            pltpu.VMEM((16,), F32),
            pltpu.SemaphoreType.DMA,
        ],
    )(x2v, winners)


def _k4_body(rows_ref, meta_ref, s2_ref, g2_ref, bb2_ref, cent_ref,
             posw_ref, posb_ref, w1_ref, b1_ref, w2_ref, b2_ref, o_ref):
    ssum = s2_ref[:, 0:1]
    ssq = s2_ref[:, 1:2]
    mean = ssum / CNT_BN
    var = ssq / CNT_BN - mean * mean
    s_tab = g2_ref[...] * jax.lax.rsqrt(var + 1e-5)
    t_tab = bb2_ref[...] - mean * s_tab
    tab = jnp.concatenate([s_tab, t_tab, jnp.zeros((C2, 6), F32)], axis=1)

    cA = meta_ref[:, 0:1]
    tsplit = meta_ref[:, 1:2]
    ccol = lax.broadcasted_iota(I32, (NTOK, C2), 1).astype(F32)
    ohA = (ccol == cA).astype(F32)
    ohB = (ccol == (cA + 1.0)).astype(F32)
    selA = jnp.dot(ohA, tab, preferred_element_type=F32)
    selB = jnp.dot(ohB, tab, preferred_element_type=F32)
    use_a = ccol < tsplit
    s_sel = jnp.where(use_a, selA[:, 0:1], selB[:, 0:1])
    t_sel = jnp.where(use_a, selA[:, 1:2], selB[:, 1:2])

    x = jnp.maximum(rows_ref[...] * s_sel + t_sel, 0.0)
    h = jnp.dot(x, w1_ref[...], preferred_element_type=F32) + b1_ref[...]
    h = 0.5 * h * (1.0 + lax.erf(h * (2.0 ** -0.5)))
    o = jnp.dot(h, w2_ref[...], preferred_element_type=F32) + b2_ref[...]

    sx = cent_ref[:, 0:1]
    sy = cent_ref[:, 1:2]
    cnt = cent_ref[:, 2:3]
    inv = jnp.where(cnt > 0.0, 1.0 / jnp.maximum(cnt, 1.0), 0.0)
    cx = sx * inv / 224.0
    cy = sy * inv / 224.0
    pos = cx * posw_ref[0:1, :] + cy * posw_ref[1:2, :] + posb_ref[...]
    o_ref[...] = o + pos


def _run_k4(rows, meta, stats2, g2, b2g, cent, pos_w, pos_b,
            l1_w, l1_b, l2_w, l2_b):
    return pl.pallas_call(
        _k4_body,
        out_shape=jax.ShapeDtypeStruct((NTOK, C2), F32),
    )(rows, meta, stats2, g2, b2g, cent, pos_w, pos_b,
      l1_w, l1_b, l2_w, l2_b)


def kernel(img, segments, conv1_w, conv1_b, bn1_g, bn1_b, conv2_w, conv2_b,
           bn2_g, bn2_b, pos_w, pos_b, l1_w, l1_b, l2_w, l2_b):
    B = img.shape[0]

    imgp = jnp.pad(img, ((0, 0), (0, 0), (3, 3), (3, 3)))
    p = jnp.stack(
        [imgp[:, :, kh:kh + 223:2, kw:kw + 223:2]
         for kh in range(7) for kw in range(7)], axis=1)
    p = p.reshape(B, 49 * 3, NPIX)
    w1r = conv1_w.transpose(0, 2, 3, 1).reshape(C1, 147)
    w2r = conv2_w.transpose(2, 3, 0, 1).reshape(9, C2, C1)
    maskrow = ((jnp.arange(PFLAT) % PW >= 1) &
               (jnp.arange(PFLAT) % PW <= WF) &
               (jnp.arange(PFLAT) >= PW) &
               (jnp.arange(PFLAT) < (HF + 1) * PW)).astype(F32)
    maskv = jnp.broadcast_to(maskrow[None, :], (8, PFLAT))
    seg_flat = segments.reshape(-1).astype(I32)
    segds_flat = segments[:, ::2, ::2].reshape(-1).astype(I32)

    x1, stats1 = _run_k1(p, w1r, conv1_b.reshape(C1, 1))

    a1p = _run_k2a(x1, stats1, bn1_g.reshape(C1, 1), bn1_b.reshape(C1, 1))
    a1f = a1p.reshape(B, C1, PFLAT)

    x2p, stats2 = _run_k2b(a1f, w2r, conv2_b.reshape(C2, 1), maskv)
    x2v = x2p.reshape(B * C2 * PH, PW)

    cent_parts, winners = _run_k3a(seg_flat, segds_flat)
    cent = (cent_parts[0] + cent_parts[1]).reshape(3, NTOKP)
    cent = jnp.concatenate(
        [cent.T[:NTOK], jnp.zeros((NTOK, 5), F32)], axis=1)

    rows, meta = _run_k3b(x2v, winners)

    out = _run_k4(rows, meta, stats2, bn2_g.reshape(C2, 1),
                  bn2_b.reshape(C2, 1), cent, pos_w, pos_b.reshape(1, C2),
                  l1_w, l1_b.reshape(1, HID), l2_w, l2_b.reshape(1, C2))
    return out.reshape(B, NSEG, C2)


# speedup vs baseline: 3.3639x; 3.3639x over previous
import functools
import jax
import jax.numpy as jnp
from jax import lax
from jax.experimental import pallas as pl
from jax.experimental.pallas import tpu as pltpu
from jax.experimental.pallas import tpu_sc as plsc

F32 = jnp.float32
I32 = jnp.int32

NSEG = 196
NTOK = 392
NTOKP = 512
C1 = 64
C2 = 384
HID = 512
HF = 112
WF = 112
NPIX = HF * WF
PH = 114
PW = 128
PFLAT = PH * PW
CNT_BN = 2.0 * NPIX


NP1 = HF * 116
SLABW = 116 * 116


def _k1_body(sl_ref, w_ref, b_ref, m_ref, x1_ref, s_ref):
    b = pl.program_id(0)
    pieces = []
    for kh in range(7):
        for kw in range(7):
            pfl = sl_ref[0, kh % 2, kw % 2]
            s = (kh // 2) * 116 + (kw // 2)
            pieces.append(pfl[:, s:s + NP1])
    big = jnp.concatenate(pieces, axis=0)
    x = jnp.dot(w_ref[...], big, preferred_element_type=F32) + b_ref[...]
    x1_ref[0] = x
    m = m_ref[0:1, :]
    ssum = jnp.sum(x * m, axis=1, keepdims=True)
    ssq = jnp.sum(x * x * m, axis=1, keepdims=True)
    blk = jnp.concatenate([ssum, ssq, jnp.zeros((C1, 126), F32)], axis=1)

    @pl.when(b == 0)
    def _():
        s_ref[...] = blk

    @pl.when(b != 0)
    def _():
        s_ref[...] = s_ref[...] + blk


def _run_k1(slabs, w1r, b1, mask1):
    B = slabs.shape[0]
    return pl.pallas_call(
        _k1_body,
        grid=(B,),
        in_specs=[
            pl.BlockSpec((1, 2, 2, 3, SLABW), lambda b: (b, 0, 0, 0, 0)),
            pl.BlockSpec((C1, 147), lambda b: (0, 0)),
            pl.BlockSpec((C1, 1), lambda b: (0, 0)),
            pl.BlockSpec((8, NP1), lambda b: (0, 0)),
        ],
        out_specs=[
            pl.BlockSpec((1, C1, NP1), lambda b: (b, 0, 0)),
            pl.BlockSpec((C1, 128), lambda b: (0, 0)),
        ],
        out_shape=[
            jax.ShapeDtypeStruct((B, C1, NP1), F32),
            jax.ShapeDtypeStruct((C1, 128), F32),
        ],
    )(slabs, w1r, b1, mask1)


def _k2a_body(x_ref, s_ref, g_ref, bb_ref, o_ref):
    ssum = s_ref[:, 0:1]
    ssq = s_ref[:, 1:2]
    mean = ssum / CNT_BN
    var = ssq / CNT_BN - mean * mean
    sc = g_ref[...] * jax.lax.rsqrt(var + 1e-5)
    sh = bb_ref[...] - mean * sc
    xv = x_ref[0][:, :, 0:WF]
    y = jnp.maximum(xv * sc[:, :, None] + sh[:, :, None], 0.0)
    zc_l = jnp.zeros((C1, HF, 1), F32)
    zc_r = jnp.zeros((C1, HF, PW - WF - 1), F32)
    zrow = jnp.zeros((C1, 1, PW), F32)
    mid = jnp.concatenate([zc_l, y, zc_r], axis=2)
    o_ref[0] = jnp.concatenate([zrow, mid, zrow], axis=1)


def _run_k2a(x1, stats1, g1, b1g):
    B = x1.shape[0]
    x1v = x1.reshape(B, C1, HF, 116)
    return pl.pallas_call(
        _k2a_body,
        grid=(B,),
        in_specs=[
            pl.BlockSpec((1, C1, HF, 116), lambda b: (b, 0, 0, 0)),
            pl.BlockSpec((C1, 128), lambda b: (0, 0)),
            pl.BlockSpec((C1, 1), lambda b: (0, 0)),
            pl.BlockSpec((C1, 1), lambda b: (0, 0)),
        ],
        out_specs=pl.BlockSpec((1, C1, PH, PW), lambda b: (b, 0, 0, 0)),
        out_shape=jax.ShapeDtypeStruct((B, C1, PH, PW), F32),
    )(x1v, stats1, g1, b1g)


_TAPS = [(dh, dw) for dh in (-1, 0, 1) for dw in (-1, 0, 1)]


def _k2b_body(a_ref, w_ref, b2_ref, m_ref, o_ref, s_ref):
    b = pl.program_id(1)
    a = a_ref[0]
    o_ref[0] = jnp.zeros((128, PFLAT), F32) + b2_ref[...]
    for t, (dh, dw) in enumerate(_TAPS):
        d = dh * PW + dw
        lo_in = max(d, 0)
        hi_in = PFLAT + min(d, 0)
        lo_o = max(-d, 0)
        hi_o = lo_o + (hi_in - lo_in)
        o_ref[0, :, lo_o:hi_o] += jnp.dot(
            w_ref[t], a[:, lo_in:hi_in], preferred_element_type=F32)
    y = o_ref[0]
    m = m_ref[0:1, :]
    ssum = jnp.sum(y * m, axis=1, keepdims=True)
    ssq = jnp.sum(y * y * m, axis=1, keepdims=True)
    blk = jnp.concatenate([ssum, ssq, jnp.zeros((128, 126), F32)], axis=1)

    @pl.when(b == 0)
    def _():
        s_ref[...] = blk

    @pl.when(b != 0)
    def _():
        s_ref[...] = s_ref[...] + blk


def _run_k2b(a1f, w2r, b2, maskv):
    B = a1f.shape[0]
    return pl.pallas_call(
        _k2b_body,
        grid=(C2 // 128, B),
        in_specs=[
            pl.BlockSpec((1, C1, PFLAT), lambda c, b: (b, 0, 0)),
            pl.BlockSpec((9, 128, C1), lambda c, b: (0, c, 0)),
            pl.BlockSpec((128, 1), lambda c, b: (c, 0)),
            pl.BlockSpec((8, PFLAT), lambda c, b: (0, 0)),
        ],
        out_specs=[
            pl.BlockSpec((1, 128, PFLAT), lambda c, b: (b, c, 0)),
            pl.BlockSpec((128, 128), lambda c, b: (c, 0)),
        ],
        out_shape=[
            jax.ShapeDtypeStruct((B, C2, PFLAT), F32),
            jax.ShapeDtypeStruct((C2, 128), F32),
        ],
    )(a1f, w2r, b2, maskv)


def _k3a_body(seg_ref, segds_ref, cent_ref, win_ref,
              segv, accv, sumv, wsegv, wtv, shared, sem):
    cid = lax.axis_index("c")
    sid = lax.axis_index("s")
    wid = sid * 2 + cid
    lanes = lax.broadcasted_iota(I32, (16,), 0)
    zero16 = jnp.zeros((16,), F32)

    def _z(i, _):
        for l in range(16):
            accv[l, pl.ds(i * 16, 16)] = zero16
        return 0
    lax.fori_loop(0, 1536 // 16, _z, 0)

    base = wid * 3136
    pltpu.sync_copy(seg_ref.at[pl.ds(base, 3136)], segv)

    def _cent(i, _):
        sv = segv[pl.ds(i * 16, 16)]
        pg = base + i * 16 + lanes
        bb = pg // (224 * 224)
        p = pg - bb * (224 * 224)
        yy = p // 224
        xx = p - yy * 224
        gid = bb * NSEG + sv
        plsc.addupdate_scatter(accv, [lanes, gid], xx.astype(F32))
        plsc.addupdate_scatter(accv, [lanes, gid + NTOKP], yy.astype(F32))
        plsc.addupdate_scatter(accv, [lanes, gid + 2 * NTOKP],
                               jnp.ones((16,), F32))
        return 0
    lax.fori_loop(0, 3136 // 16, _cent, 0)

    def _fold(i, _):
        acc = zero16
        for l in range(16):
            acc = acc + accv[l, pl.ds(i * 16, 16)]
        sumv[pl.ds(i * 16, 16)] = acc
        return 0
    lax.fori_loop(0, 1536 // 16, _fold, 0)

    @pl.when(jnp.logical_and(cid == 1, sid == 15))
    def _():
        pltpu.sync_copy(segds_ref, wsegv)
        neg = jnp.zeros((16,), I32) - 1

        def _wz(i, _):
            for l in range(16):
                wtv[l, pl.ds(i * 16, 16)] = neg
            return 0
        lax.fori_loop(0, NTOKP // 16, _wz, 0)

        def _win(i, _):
            sv = wsegv[pl.ds(i * 16, 16)]
            iflat = i * 16 + lanes
            bb = iflat // NPIX
            gid = bb * NSEG + sv
            plsc.store_scatter(wtv, [lanes, gid], iflat)
            return 0
        lax.fori_loop(0, (2 * NPIX) // 16, _win, 0)

        def _wfold(i, _):
            acc = neg
            for l in range(16):
                acc = jnp.maximum(acc, wtv[l, pl.ds(i * 16, 16)])
            wsegv[pl.ds(i * 16, 16)] = acc
            return 0
        lax.fori_loop(0, NTOKP // 16, _wfold, 0)
        pltpu.sync_copy(wsegv.at[pl.ds(0, NTOKP)], win_ref)

    pltpu.sync_copy(sumv, shared.at[sid])
    plsc.subcore_barrier()

    @pl.when(sid == 0)
    def _():
        pltpu.sync_copy(shared, accv)

        def _red(i, _):
            acc = zero16
            for s in range(16):
                acc = acc + accv[s, pl.ds(i * 16, 16)]
            sumv[pl.ds(i * 16, 16)] = acc
            return 0
        lax.fori_loop(0, 1536 // 16, _red, 0)
        pltpu.sync_copy(sumv, cent_ref.at[cid])


def _run_k3a(seg_flat, segds_flat):
    mesh = plsc.VectorSubcoreMesh(core_axis_name="c", subcore_axis_name="s")
    return pl.kernel(
        _k3a_body,
        compiler_params=pltpu.CompilerParams(needs_layout_passes=False),
        out_type=[
            jax.ShapeDtypeStruct((2, 1536), F32),
            jax.ShapeDtypeStruct((NTOKP,), I32),
        ],
        mesh=mesh,
        scratch_types=[
            pltpu.VMEM((3136,), I32),
            pltpu.VMEM((16, 1536), F32),
            pltpu.VMEM((1536,), F32),
            pltpu.VMEM((2 * NPIX,), I32),
            pltpu.VMEM((16, NTOKP), I32),
            pltpu.VMEM_SHARED((16, 1536), F32),
            pltpu.SemaphoreType.DMA,
        ],
    )(seg_flat, segds_flat)


def _k3b_body(x2v_ref, win_ref, rows_ref, meta_ref,
              winv, staged, rowbuf, metabuf, sem):
    cid = lax.axis_index("c")
    sid = lax.axis_index("s")
    wid = sid * 2 + cid
    lanes = lax.broadcasted_iota(I32, (16,), 0)
    pltpu.sync_copy(win_ref, winv)

    def _row(k, _):
        r = (wid + k * 32) % NTOK
        i = plsc.load_gather(winv, [jnp.zeros((16,), I32) + r])
        bb = i // NPIX
        n = i - bb * NPIX
        q0 = n * C2
        c20 = q0 // NPIX
        p0 = q0 - c20 * NPIX
        w0 = p0 % WF
        qs = q0 + jnp.minimum(
            jnp.maximum((WF - w0) + (lanes - 1) * WF, 0), C2 - 1)
        c2s = qs // NPIX
        ps = qs - c2s * NPIX
        hs = ps // WF
        hrow = (bb * C2 + c2s) * PH + hs + 1
        pltpu.async_copy(x2v_ref.at[hrow], staged, sem).wait()

        def _pack(j24, _):
            jv = j24 * 16 + lanes
            slot = (w0 + jv) // WF
            wj = (w0 + jv) - slot * WF
            rowbuf[pl.ds(j24 * 16, 16)] = plsc.load_gather(
                staged, [slot, wj + 1])
            return 0
        lax.fori_loop(0, C2 // 16, _pack, 0)

        pltpu.sync_copy(rowbuf, rows_ref.at[r])
        tsplit = (c20 + 1) * NPIX - q0
        mv = (jnp.where(lanes == 0, c20, 0) +
              jnp.where(lanes == 1, tsplit, 0)).astype(F32)
        metabuf[...] = mv
        pltpu.sync_copy(metabuf, meta_ref.at[r])
        return 0
    lax.fori_loop(0, 13, _row, 0)


def _run_k3b(x2v, winners):
    mesh = plsc.VectorSubcoreMesh(core_axis_name="c", subcore_axis_name="s")
    return pl.kernel(
        _k3b_body,
        compiler_params=pltpu.CompilerParams(needs_layout_passes=False),
        out_type=[
            jax.ShapeDtypeStruct((NTOK, C2), F32),
            jax.ShapeDtypeStruct((NTOK, 16), F32),
        ],
        mesh=mesh,
        scratch_types=[
            pltpu.VMEM((NTOKP,), I32),
            pltpu.VMEM((16, PW), F32),
            pltpu.VMEM((C2,), F32),
            pltpu.VMEM((16,), F32),
            pltpu.SemaphoreType.DMA,
        ],
    )(x2v, winners)


def _k4_body(rows_ref, meta_ref, s2_ref, g2_ref, bb2_ref, cent_ref,
             posw_ref, posb_ref, w1_ref, b1_ref, w2_ref, b2_ref, o_ref):
    ssum = s2_ref[:, 0:1]
    ssq = s2_ref[:, 1:2]
    mean = ssum / CNT_BN
    var = ssq / CNT_BN - mean * mean
    s_tab = g2_ref[...] * jax.lax.rsqrt(var + 1e-5)
    t_tab = bb2_ref[...] - mean * s_tab
    tab = jnp.concatenate([s_tab, t_tab, jnp.zeros((C2, 6), F32)], axis=1)

    cA = meta_ref[:, 0:1]
    tsplit = meta_ref[:, 1:2]
    ccol = lax.broadcasted_iota(I32, (NTOK, C2), 1).astype(F32)
    ohA = (ccol == cA).astype(F32)
    ohB = (ccol == (cA + 1.0)).astype(F32)
    selA = jnp.dot(ohA, tab, preferred_element_type=F32)
    selB = jnp.dot(ohB, tab, preferred_element_type=F32)
    use_a = ccol < tsplit
    s_sel = jnp.where(use_a, selA[:, 0:1], selB[:, 0:1])
    t_sel = jnp.where(use_a, selA[:, 1:2], selB[:, 1:2])

    x = jnp.maximum(rows_ref[...] * s_sel + t_sel, 0.0)
    h = jnp.dot(x, w1_ref[...], preferred_element_type=F32) + b1_ref[...]
    h = 0.5 * h * (1.0 + lax.erf(h * (2.0 ** -0.5)))
    o = jnp.dot(h, w2_ref[...], preferred_element_type=F32) + b2_ref[...]

    sx = cent_ref[:, 0:1]
    sy = cent_ref[:, 1:2]
    cnt = cent_ref[:, 2:3]
    inv = jnp.where(cnt > 0.0, 1.0 / jnp.maximum(cnt, 1.0), 0.0)
    cx = sx * inv / 224.0
    cy = sy * inv / 224.0
    pos = cx * posw_ref[0:1, :] + cy * posw_ref[1:2, :] + posb_ref[...]
    o_ref[...] = o + pos


def _run_k4(rows, meta, stats2, g2, b2g, cent, pos_w, pos_b,
            l1_w, l1_b, l2_w, l2_b):
    return pl.pallas_call(
        _k4_body,
        out_shape=jax.ShapeDtypeStruct((NTOK, C2), F32),
    )(rows, meta, stats2, g2, b2g, cent, pos_w, pos_b,
      l1_w, l1_b, l2_w, l2_b)


def kernel(img, segments, conv1_w, conv1_b, bn1_g, bn1_b, conv2_w, conv2_b,
           bn2_g, bn2_b, pos_w, pos_b, l1_w, l1_b, l2_w, l2_b):
    B = img.shape[0]

    imgp = jnp.pad(img, ((0, 0), (0, 0), (3, 5), (3, 5)))
    slabs = imgp.reshape(B, 3, 116, 2, 116, 2).transpose(0, 3, 5, 1, 2, 4)
    slabs = slabs.reshape(B, 2, 2, 3, SLABW)
    w1r = conv1_w.transpose(2, 3, 1, 0).reshape(147, C1).T
    mask1row = (jnp.arange(NP1) % 116 < WF).astype(F32)
    mask1 = jnp.broadcast_to(mask1row[None, :], (8, NP1))
    w2r = conv2_w.transpose(2, 3, 0, 1).reshape(9, C2, C1)
    maskrow = ((jnp.arange(PFLAT) % PW >= 1) &
               (jnp.arange(PFLAT) % PW <= WF) &
               (jnp.arange(PFLAT) >= PW) &
               (jnp.arange(PFLAT) < (HF + 1) * PW)).astype(F32)
    maskv = jnp.broadcast_to(maskrow[None, :], (8, PFLAT))
    seg_flat = segments.reshape(-1).astype(I32)
    segds_flat = segments[:, ::2, ::2].reshape(-1).astype(I32)

    x1, stats1 = _run_k1(slabs, w1r, conv1_b.reshape(C1, 1), mask1)

    a1p = _run_k2a(x1, stats1, bn1_g.reshape(C1, 1), bn1_b.reshape(C1, 1))
    a1f = a1p.reshape(B, C1, PFLAT)

    x2p, stats2 = _run_k2b(a1f, w2r, conv2_b.reshape(C2, 1), maskv)
    x2v = x2p.reshape(B * C2 * PH, PW)

    cent_parts, winners = _run_k3a(seg_flat, segds_flat)
    cent = (cent_parts[0] + cent_parts[1]).reshape(3, NTOKP)
    cent = jnp.concatenate(
        [cent.T[:NTOK], jnp.zeros((NTOK, 5), F32)], axis=1)

    rows, meta = _run_k3b(x2v, winners)

    out = _run_k4(rows, meta, stats2, bn2_g.reshape(C2, 1),
                  bn2_b.reshape(C2, 1), cent, pos_w, pos_b.reshape(1, C2),
                  l1_w, l1_b.reshape(1, HID), l2_w, l2_b.reshape(1, C2))
    return out.reshape(B, NSEG, C2)


# speedup vs baseline: 3.3902x; 1.0078x over previous
import functools
import jax
import jax.numpy as jnp
from jax import lax
from jax.experimental import pallas as pl
from jax.experimental.pallas import tpu as pltpu
from jax.experimental.pallas import tpu_sc as plsc

F32 = jnp.float32
I32 = jnp.int32

NSEG = 196
NTOK = 392
NTOKP = 512
C1 = 64
C2 = 384
HID = 512
HF = 112
WF = 112
NPIX = HF * WF
PH = 114
PW = 128
PFLAT = PH * PW
CNT_BN = 2.0 * NPIX


NP1 = HF * 116
SLABW = 116 * 116


def _k1_body(sl_ref, w_ref, b_ref, m_ref, x1_ref, s_ref):
    b = pl.program_id(0)
    pieces = []
    for kh in range(7):
        for kw in range(7):
            pfl = sl_ref[0, kh % 2, kw % 2]
            s = (kh // 2) * 116 + (kw // 2)
            pieces.append(pfl[:, s:s + NP1])
    big = jnp.concatenate(pieces, axis=0)
    x = jnp.dot(w_ref[...], big, preferred_element_type=F32) + b_ref[...]
    x1_ref[0] = x
    m = m_ref[0:1, :]
    ssum = jnp.sum(x * m, axis=1, keepdims=True)
    ssq = jnp.sum(x * x * m, axis=1, keepdims=True)
    blk = jnp.concatenate([ssum, ssq, jnp.zeros((C1, 126), F32)], axis=1)

    @pl.when(b == 0)
    def _():
        s_ref[...] = blk

    @pl.when(b != 0)
    def _():
        s_ref[...] = s_ref[...] + blk


def _run_k1(slabs, w1r, b1, mask1):
    B = slabs.shape[0]
    return pl.pallas_call(
        _k1_body,
        grid=(B,),
        in_specs=[
            pl.BlockSpec((1, 2, 2, 3, SLABW), lambda b: (b, 0, 0, 0, 0)),
            pl.BlockSpec((C1, 147), lambda b: (0, 0)),
            pl.BlockSpec((C1, 1), lambda b: (0, 0)),
            pl.BlockSpec((8, NP1), lambda b: (0, 0)),
        ],
        out_specs=[
            pl.BlockSpec((1, C1, NP1), lambda b: (b, 0, 0)),
            pl.BlockSpec((C1, 128), lambda b: (0, 0)),
        ],
        out_shape=[
            jax.ShapeDtypeStruct((B, C1, NP1), F32),
            jax.ShapeDtypeStruct((C1, 128), F32),
        ],
    )(slabs, w1r, b1, mask1)


def _k2a_body(x_ref, s_ref, g_ref, bb_ref, o_ref):
    ssum = s_ref[:, 0:1]
    ssq = s_ref[:, 1:2]
    mean = ssum / CNT_BN
    var = ssq / CNT_BN - mean * mean
    sc = g_ref[...] * jax.lax.rsqrt(var + 1e-5)
    sh = bb_ref[...] - mean * sc
    xv = x_ref[0][:, :, 0:WF]
    y = jnp.maximum(xv * sc[:, :, None] + sh[:, :, None], 0.0)
    zc_l = jnp.zeros((C1, HF, 1), F32)
    zc_r = jnp.zeros((C1, HF, PW - WF - 1), F32)
    zrow = jnp.zeros((C1, 1, PW), F32)
    mid = jnp.concatenate([zc_l, y, zc_r], axis=2)
    o_ref[0] = jnp.concatenate([zrow, mid, zrow], axis=1)


def _run_k2a(x1, stats1, g1, b1g):
    B = x1.shape[0]
    x1v = x1.reshape(B, C1, HF, 116)
    return pl.pallas_call(
        _k2a_body,
        grid=(B,),
        in_specs=[
            pl.BlockSpec((1, C1, HF, 116), lambda b: (b, 0, 0, 0)),
            pl.BlockSpec((C1, 128), lambda b: (0, 0)),
            pl.BlockSpec((C1, 1), lambda b: (0, 0)),
            pl.BlockSpec((C1, 1), lambda b: (0, 0)),
        ],
        out_specs=pl.BlockSpec((1, C1, PH, PW), lambda b: (b, 0, 0, 0)),
        out_shape=jax.ShapeDtypeStruct((B, C1, PH, PW), F32),
    )(x1v, stats1, g1, b1g)


_TAPS = [(dh, dw) for dh in (-1, 0, 1) for dw in (-1, 0, 1)]


def _k2b_body(a_ref, w_ref, b2_ref, m_ref, o_ref, s_ref):
    b = pl.program_id(1)
    a = a_ref[0].astype(jnp.bfloat16)
    o_ref[0] = jnp.zeros((128, PFLAT), F32) + b2_ref[...]
    for t, (dh, dw) in enumerate(_TAPS):
        d = dh * PW + dw
        lo_in = max(d, 0)
        hi_in = PFLAT + min(d, 0)
        lo_o = max(-d, 0)
        hi_o = lo_o + (hi_in - lo_in)
        o_ref[0, :, lo_o:hi_o] += jnp.dot(
            w_ref[t], a[:, lo_in:hi_in], preferred_element_type=F32)
    y = o_ref[0]
    m = m_ref[0:1, :]
    ssum = jnp.sum(y * m, axis=1, keepdims=True)
    ssq = jnp.sum(y * y * m, axis=1, keepdims=True)
    blk = jnp.concatenate([ssum, ssq, jnp.zeros((128, 126), F32)], axis=1)

    @pl.when(b == 0)
    def _():
        s_ref[...] = blk

    @pl.when(b != 0)
    def _():
        s_ref[...] = s_ref[...] + blk


def _run_k2b(a1f, w2r, b2, maskv):
    B = a1f.shape[0]
    return pl.pallas_call(
        _k2b_body,
        grid=(C2 // 128, B),
        in_specs=[
            pl.BlockSpec((1, C1, PFLAT), lambda c, b: (b, 0, 0)),
            pl.BlockSpec((9, 128, C1), lambda c, b: (0, c, 0)),
            pl.BlockSpec((128, 1), lambda c, b: (c, 0)),
            pl.BlockSpec((8, PFLAT), lambda c, b: (0, 0)),
        ],
        out_specs=[
            pl.BlockSpec((1, 128, PFLAT), lambda c, b: (b, c, 0)),
            pl.BlockSpec((128, 128), lambda c, b: (c, 0)),
        ],
        out_shape=[
            jax.ShapeDtypeStruct((B, C2, PFLAT), F32),
            jax.ShapeDtypeStruct((C2, 128), F32),
        ],
    )(a1f, w2r, b2, maskv)


def _k3a_body(seg_ref, segds_ref, cent_ref, win_ref,
              segv, accv, sumv, wsegv, wtv, shared, sem):
    cid = lax.axis_index("c")
    sid = lax.axis_index("s")
    wid = sid * 2 + cid
    lanes = lax.broadcasted_iota(I32, (16,), 0)
    zero16 = jnp.zeros((16,), F32)

    def _z(i, _):
        for l in range(16):
            accv[l, pl.ds(i * 16, 16)] = zero16
        return 0
    lax.fori_loop(0, 1536 // 16, _z, 0)

    base = wid * 3136
    pltpu.sync_copy(seg_ref.at[pl.ds(base, 3136)], segv)

    def _cent(i, _):
        sv = segv[pl.ds(i * 16, 16)]
        pg = base + i * 16 + lanes
        bb = pg // (224 * 224)
        p = pg - bb * (224 * 224)
        yy = p // 224
        xx = p - yy * 224
        gid = bb * NSEG + sv
        plsc.addupdate_scatter(accv, [lanes, gid], xx.astype(F32))
        plsc.addupdate_scatter(accv, [lanes, gid + NTOKP], yy.astype(F32))
        plsc.addupdate_scatter(accv, [lanes, gid + 2 * NTOKP],
                               jnp.ones((16,), F32))
        return 0
    lax.fori_loop(0, 3136 // 16, _cent, 0)

    def _fold(i, _):
        acc = zero16
        for l in range(16):
            acc = acc + accv[l, pl.ds(i * 16, 16)]
        sumv[pl.ds(i * 16, 16)] = acc
        return 0
    lax.fori_loop(0, 1536 // 16, _fold, 0)

    @pl.when(jnp.logical_and(cid == 1, sid == 15))
    def _():
        pltpu.sync_copy(segds_ref, wsegv)
        neg = jnp.zeros((16,), I32) - 1

        def _wz(i, _):
            for l in range(16):
                wtv[l, pl.ds(i * 16, 16)] = neg
            return 0
        lax.fori_loop(0, NTOKP // 16, _wz, 0)

        def _win(i, _):
            sv = wsegv[pl.ds(i * 16, 16)]
            iflat = i * 16 + lanes
            bb = iflat // NPIX
            gid = bb * NSEG + sv
            plsc.store_scatter(wtv, [lanes, gid], iflat)
            return 0
        lax.fori_loop(0, (2 * NPIX) // 16, _win, 0)

        def _wfold(i, _):
            acc = neg
            for l in range(16):
                acc = jnp.maximum(acc, wtv[l, pl.ds(i * 16, 16)])
            wsegv[pl.ds(i * 16, 16)] = acc
            return 0
        lax.fori_loop(0, NTOKP // 16, _wfold, 0)
        pltpu.sync_copy(wsegv.at[pl.ds(0, NTOKP)], win_ref)

    pltpu.sync_copy(sumv, shared.at[sid])
    plsc.subcore_barrier()

    @pl.when(sid == 0)
    def _():
        pltpu.sync_copy(shared, accv)

        def _red(i, _):
            acc = zero16
            for s in range(16):
                acc = acc + accv[s, pl.ds(i * 16, 16)]
            sumv[pl.ds(i * 16, 16)] = acc
            return 0
        lax.fori_loop(0, 1536 // 16, _red, 0)
        pltpu.sync_copy(sumv, cent_ref.at[cid])


def _run_k3a(seg_flat, segds_flat):
    mesh = plsc.VectorSubcoreMesh(core_axis_name="c", subcore_axis_name="s")
    return pl.kernel(
        _k3a_body,
        compiler_params=pltpu.CompilerParams(needs_layout_passes=False),
        out_type=[
            jax.ShapeDtypeStruct((2, 1536), F32),
            jax.ShapeDtypeStruct((NTOKP,), I32),
        ],
        mesh=mesh,
        scratch_types=[
            pltpu.VMEM((3136,), I32),
            pltpu.VMEM((16, 1536), F32),
            pltpu.VMEM((1536,), F32),
            pltpu.VMEM((2 * NPIX,), I32),
            pltpu.VMEM((16, NTOKP), I32),
            pltpu.VMEM_SHARED((16, 1536), F32),
            pltpu.SemaphoreType.DMA,
        ],
    )(seg_flat, segds_flat)


def _k3b_body(x2v_ref, win_ref, rows_ref, meta_ref,
              winv, staged, rowbuf, metabuf, sem):
    cid = lax.axis_index("c")
    sid = lax.axis_index("s")
    wid = sid * 2 + cid
    lanes = lax.broadcasted_iota(I32, (16,), 0)
    pltpu.sync_copy(win_ref, winv)

    def _row(k, _):
        r = (wid + k * 32) % NTOK
        i = plsc.load_gather(winv, [jnp.zeros((16,), I32) + r])
        bb = i // NPIX
        n = i - bb * NPIX
        q0 = n * C2
        c20 = q0 // NPIX
        p0 = q0 - c20 * NPIX
        w0 = p0 % WF
        qs = q0 + jnp.minimum(
            jnp.maximum((WF - w0) + (lanes - 1) * WF, 0), C2 - 1)
        c2s = qs // NPIX
        ps = qs - c2s * NPIX
        hs = ps // WF
        hrow = (bb * C2 + c2s) * PH + hs + 1
        pltpu.async_copy(x2v_ref.at[hrow], staged, sem).wait()

        def _pack(j24, _):
            jv = j24 * 16 + lanes
            slot = (w0 + jv) // WF
            wj = (w0 + jv) - slot * WF
            rowbuf[pl.ds(j24 * 16, 16)] = plsc.load_gather(
                staged, [slot, wj + 1])
            return 0
        lax.fori_loop(0, C2 // 16, _pack, 0)

        pltpu.sync_copy(rowbuf, rows_ref.at[r])
        tsplit = (c20 + 1) * NPIX - q0
        mv = (jnp.where(lanes == 0, c20, 0) +
              jnp.where(lanes == 1, tsplit, 0)).astype(F32)
        metabuf[...] = mv
        pltpu.sync_copy(metabuf, meta_ref.at[r])
        return 0
    lax.fori_loop(0, 13, _row, 0)


def _run_k3b(x2v, winners):
    mesh = plsc.VectorSubcoreMesh(core_axis_name="c", subcore_axis_name="s")
    return pl.kernel(
        _k3b_body,
        compiler_params=pltpu.CompilerParams(needs_layout_passes=False),
        out_type=[
            jax.ShapeDtypeStruct((NTOK, C2), F32),
            jax.ShapeDtypeStruct((NTOK, 16), F32),
        ],
        mesh=mesh,
        scratch_types=[
            pltpu.VMEM((NTOKP,), I32),
            pltpu.VMEM((16, PW), F32),
            pltpu.VMEM((C2,), F32),
            pltpu.VMEM((16,), F32),
            pltpu.SemaphoreType.DMA,
        ],
    )(x2v, winners)


def _k4_body(rows_ref, meta_ref, s2_ref, g2_ref, bb2_ref, cent_ref,
             posw_ref, posb_ref, w1_ref, b1_ref, w2_ref, b2_ref, o_ref):
    ssum = s2_ref[:, 0:1]
    ssq = s2_ref[:, 1:2]
    mean = ssum / CNT_BN
    var = ssq / CNT_BN - mean * mean
    s_tab = g2_ref[...] * jax.lax.rsqrt(var + 1e-5)
    t_tab = bb2_ref[...] - mean * s_tab
    tab = jnp.concatenate([s_tab, t_tab, jnp.zeros((C2, 6), F32)], axis=1)

    cA = meta_ref[:, 0:1]
    tsplit = meta_ref[:, 1:2]
    ccol = lax.broadcasted_iota(I32, (NTOK, C2), 1).astype(F32)
    ohA = (ccol == cA).astype(F32)
    ohB = (ccol == (cA + 1.0)).astype(F32)
    selA = jnp.dot(ohA, tab, preferred_element_type=F32)
    selB = jnp.dot(ohB, tab, preferred_element_type=F32)
    use_a = ccol < tsplit
    s_sel = jnp.where(use_a, selA[:, 0:1], selB[:, 0:1])
    t_sel = jnp.where(use_a, selA[:, 1:2], selB[:, 1:2])

    x = jnp.maximum(rows_ref[...] * s_sel + t_sel, 0.0)
    h = jnp.dot(x, w1_ref[...], preferred_element_type=F32) + b1_ref[...]
    h = 0.5 * h * (1.0 + lax.erf(h * (2.0 ** -0.5)))
    o = jnp.dot(h, w2_ref[...], preferred_element_type=F32) + b2_ref[...]

    sx = cent_ref[:, 0:1]
    sy = cent_ref[:, 1:2]
    cnt = cent_ref[:, 2:3]
    inv = jnp.where(cnt > 0.0, 1.0 / jnp.maximum(cnt, 1.0), 0.0)
    cx = sx * inv / 224.0
    cy = sy * inv / 224.0
    pos = cx * posw_ref[0:1, :] + cy * posw_ref[1:2, :] + posb_ref[...]
    o_ref[...] = o + pos


def _run_k4(rows, meta, stats2, g2, b2g, cent, pos_w, pos_b,
            l1_w, l1_b, l2_w, l2_b):
    return pl.pallas_call(
        _k4_body,
        out_shape=jax.ShapeDtypeStruct((NTOK, C2), F32),
    )(rows, meta, stats2, g2, b2g, cent, pos_w, pos_b,
      l1_w, l1_b, l2_w, l2_b)


def kernel(img, segments, conv1_w, conv1_b, bn1_g, bn1_b, conv2_w, conv2_b,
           bn2_g, bn2_b, pos_w, pos_b, l1_w, l1_b, l2_w, l2_b):
    B = img.shape[0]

    imgp = jnp.pad(img, ((0, 0), (0, 0), (3, 5), (3, 5)))
    slabs = imgp.reshape(B, 3, 116, 2, 116, 2).transpose(0, 3, 5, 1, 2, 4)
    slabs = slabs.reshape(B, 2, 2, 3, SLABW)
    w1r = conv1_w.transpose(2, 3, 1, 0).reshape(147, C1).T
    mask1row = (jnp.arange(NP1) % 116 < WF).astype(F32)
    mask1 = jnp.broadcast_to(mask1row[None, :], (8, NP1))
    w2r = conv2_w.transpose(2, 3, 0, 1).reshape(9, C2, C1).astype(jnp.bfloat16)
    maskrow = ((jnp.arange(PFLAT) % PW >= 1) &
               (jnp.arange(PFLAT) % PW <= WF) &
               (jnp.arange(PFLAT) >= PW) &
               (jnp.arange(PFLAT) < (HF + 1) * PW)).astype(F32)
    maskv = jnp.broadcast_to(maskrow[None, :], (8, PFLAT))
    seg_flat = segments.reshape(-1).astype(I32)
    segds_flat = segments[:, ::2, ::2].reshape(-1).astype(I32)

    x1, stats1 = _run_k1(slabs, w1r, conv1_b.reshape(C1, 1), mask1)

    a1p = _run_k2a(x1, stats1, bn1_g.reshape(C1, 1), bn1_b.reshape(C1, 1))
    a1f = a1p.reshape(B, C1, PFLAT)

    x2p, stats2 = _run_k2b(a1f, w2r, conv2_b.reshape(C2, 1), maskv)
    x2v = x2p.reshape(B * C2 * PH, PW)

    cent_parts, winners = _run_k3a(seg_flat, segds_flat)
    cent = (cent_parts[0] + cent_parts[1]).reshape(3, NTOKP)
    cent = jnp.concatenate(
        [cent.T[:NTOK], jnp.zeros((NTOK, 5), F32)], axis=1)

    rows, meta = _run_k3b(x2v, winners)

    out = _run_k4(rows, meta, stats2, bn2_g.reshape(C2, 1),
                  bn2_b.reshape(C2, 1), cent, pos_w, pos_b.reshape(1, C2),
                  l1_w, l1_b.reshape(1, HID), l2_w, l2_b.reshape(1, C2))
    return out.reshape(B, NSEG, C2)
